# baseline (speedup 1.0000x reference)
import jax
import jax.numpy as jnp
from jax.experimental import pallas as pl
from jax.experimental.pallas import tpu as pltpu

_NCHUNK = 8


def _dma_copy(t_ref, o_ref, *scratch):
    bufs = scratch[:_NCHUNK]
    rsem = scratch[_NCHUNK]
    wsem = scratch[_NCHUNK + 1]
    rows = t_ref.shape[0]
    c = rows // _NCHUNK
    for i in range(_NCHUNK):
        pltpu.make_async_copy(
            t_ref.at[pl.ds(i * c, c)], bufs[i], rsem.at[i]
        ).start()
    for i in range(_NCHUNK):
        pltpu.make_async_copy(
            t_ref.at[pl.ds(i * c, c)], bufs[i], rsem.at[i]
        ).wait()
        pltpu.make_async_copy(
            bufs[i], o_ref.at[pl.ds(i * c, c)], wsem.at[i]
        ).start()
    for i in range(_NCHUNK):
        pltpu.make_async_copy(
            bufs[i], o_ref.at[pl.ds(i * c, c)], wsem.at[i]
        ).wait()


def kernel(x, table):
    del x
    T, E = table.shape
    ch = T // _NCHUNK
    out = pl.pallas_call(
        _dma_copy,
        in_specs=[pl.BlockSpec(memory_space=pl.ANY)],
        out_specs=pl.BlockSpec(memory_space=pl.ANY),
        out_shape=jax.ShapeDtypeStruct((T, E), table.dtype),
        scratch_shapes=(
            [pltpu.VMEM((ch, E), table.dtype) for _ in range(_NCHUNK)]
            + [pltpu.SemaphoreType.DMA((_NCHUNK,)),
               pltpu.SemaphoreType.DMA((_NCHUNK,))]
        ),
    )(table)
    return out[None, :, :]


# speedup vs baseline: 1.0212x; 1.0212x over previous
import jax
import jax.numpy as jnp
from jax.experimental import pallas as pl

_ROWS = 1024


def _copy_block(t_ref, o_ref):
    o_ref[...] = t_ref[...]


def kernel(x, table):
    del x
    T, E = table.shape
    out = pl.pallas_call(
        _copy_block,
        grid=(T // _ROWS,),
        in_specs=[pl.BlockSpec((_ROWS, E), lambda i: (i, 0))],
        out_specs=pl.BlockSpec((_ROWS, E), lambda i: (i, 0)),
        out_shape=jax.ShapeDtypeStruct((T, E), table.dtype),
    )(table)
    return out[None, :, :]
